# baseline (speedup 1.0000x reference)
import functools
import operator

import jax
import jax.numpy as jnp
from jax.experimental import pallas as pl
from jax.experimental.pallas import tpu as pltpu

B, F_IN, F_OUT = 64, 4096, 4096
OB = 512
PG = OB // 32


def _mm_body(x_ref, w_ref, m_ref, o_ref):
    mp = m_ref[...]
    sh = 31 - jax.lax.broadcasted_iota(jnp.int32, (PG, 32, F_IN), 1)
    t = jax.lax.shift_left(mp[:, None, :], sh)
    neg = jax.lax.shift_right_arithmetic(t, 31)
    wi = jax.lax.bitcast_convert_type(w_ref[...], jnp.int32)
    wm = jax.lax.bitcast_convert_type(
        (wi.reshape(PG, 32, F_IN) & neg).reshape(OB, F_IN), jnp.float32)
    o_ref[...] = jax.lax.dot_general(
        x_ref[...], wm, (((1,), (1,)), ((), ())),
        preferred_element_type=jnp.float32)


def kernel(x, weight, mask):
    mr = mask.reshape(F_OUT // 32, 32, F_IN)
    mp = functools.reduce(
        operator.or_,
        (mr[:, b, :].astype(jnp.int32) << b for b in range(32)))
    grid = (F_OUT // OB,)
    return pl.pallas_call(
        _mm_body,
        grid=grid,
        in_specs=[
            pl.BlockSpec((B, F_IN), lambda o: (0, 0)),
            pl.BlockSpec((OB, F_IN), lambda o: (o, 0)),
            pl.BlockSpec((PG, F_IN), lambda o: (o, 0)),
        ],
        out_specs=pl.BlockSpec((B, OB), lambda o: (0, o)),
        out_shape=jax.ShapeDtypeStruct((B, F_OUT), jnp.float32),
        compiler_params=pltpu.CompilerParams(
            dimension_semantics=("arbitrary",)),
    )(x, weight, mp)


# speedup vs baseline: 6.5477x; 6.5477x over previous
import jax
import jax.numpy as jnp
from jax.experimental import pallas as pl
from jax.experimental.pallas import tpu as pltpu

B, F_IN, F_OUT = 64, 4096, 4096
OB = 512


def _mm_body(x_ref, w_ref, m_ref, o_ref):
    wm = w_ref[...] * m_ref[...].astype(jnp.float32)
    o_ref[...] = jax.lax.dot_general(
        x_ref[...], wm, (((1,), (1,)), ((), ())),
        preferred_element_type=jnp.float32)


def kernel(x, weight, mask):
    m4 = mask.astype(jnp.int4)
    grid = (F_OUT // OB,)
    return pl.pallas_call(
        _mm_body,
        grid=grid,
        in_specs=[
            pl.BlockSpec((B, F_IN), lambda o: (0, 0)),
            pl.BlockSpec((OB, F_IN), lambda o: (o, 0)),
            pl.BlockSpec((OB, F_IN), lambda o: (o, 0)),
        ],
        out_specs=pl.BlockSpec((B, OB), lambda o: (0, o)),
        out_shape=jax.ShapeDtypeStruct((B, F_OUT), jnp.float32),
        compiler_params=pltpu.CompilerParams(
            dimension_semantics=("arbitrary",),
            allow_input_fusion=[False, False, True]),
    )(x, weight, m4)
